# baseline (speedup 1.0000x reference)
import jax
import jax.numpy as jnp
from jax import lax
from jax.experimental import pallas as pl
from jax.experimental.pallas import tpu as pltpu
from jax.experimental.pallas import tpu_sc as plsc

_CH = 128
_NG = 1
_NB = 4


def kernel(x, emb_weight):
    B, L = x.shape
    N = B * L
    V, D = emb_weight.shape
    idx = x.reshape(N).astype(jnp.int32)

    NW = 32
    per_w = N // NW
    steps = per_w // _CH
    mesh = plsc.VectorSubcoreMesh(core_axis_name="core", subcore_axis_name="subcore")

    @pl.kernel(
        out_type=jax.ShapeDtypeStruct((N, D), emb_weight.dtype),
        mesh=mesh,
        scratch_types=(
            [pltpu.VMEM((per_w,), jnp.int32)]
            + [pltpu.VMEM((_CH, D), jnp.float32) for _ in range(_NB)]
            + [pltpu.SemaphoreType.DMA for _ in range(_NG * _NB + _NB)]
        ),
    )
    def run(table_hbm, idx_hbm, out_hbm, idx_v, *scratch):
        bufs = scratch[:_NB]
        gsem = scratch[_NB:_NB + _NG * _NB]
        wsem = scratch[_NB + _NG * _NB:]
        wid = lax.axis_index("subcore") * 2 + lax.axis_index("core")
        base = wid * per_w

        pltpu.sync_copy(idx_hbm.at[pl.ds(base, per_w)], idx_v)

        def gathers(g, b):
            return [
                pltpu.make_async_copy(
                    table_hbm.at[idx_v.at[pl.ds(g * _CH, _CH)]],
                    bufs[b],
                    gsem[b],
                )
            ]

        def write(g, b):
            return pltpu.make_async_copy(
                bufs[b], out_hbm.at[pl.ds(base + g * _CH, _CH)], wsem[b])

        for c in gathers(0, 0):
            c.start()

        @pl.loop(0, steps, step=_NB)
        def _(g0):
            for b in range(_NB):
                g = g0 + b
                bn = (b + 1) % _NB

                @pl.when(jnp.logical_and(g + 1 >= _NB, g + 1 < steps))
                def _():
                    write(g + 1 - _NB, bn).wait()

                @pl.when(g + 1 < steps)
                def _():
                    for c in gathers(g + 1, bn):
                        c.start()

                for c in gathers(g, b):
                    c.wait()
                write(g, b).start()

        for k in range(_NB):
            w = steps - _NB + k
            write(w, w % _NB).wait()

    return run(emb_weight, idx).reshape(B, L, D)


# speedup vs baseline: 1.0040x; 1.0040x over previous
import jax
import jax.numpy as jnp
from jax import lax
from jax.experimental import pallas as pl
from jax.experimental.pallas import tpu as pltpu
from jax.experimental.pallas import tpu_sc as plsc

_CH = 128
_NG = 1
_NB = 5


def kernel(x, emb_weight):
    B, L = x.shape
    N = B * L
    V, D = emb_weight.shape
    idx = x.reshape(N).astype(jnp.int32)

    NW = 32
    per_w = N // NW
    steps = per_w // _CH
    mesh = plsc.VectorSubcoreMesh(core_axis_name="core", subcore_axis_name="subcore")

    @pl.kernel(
        out_type=jax.ShapeDtypeStruct((N, D), emb_weight.dtype),
        mesh=mesh,
        scratch_types=(
            [pltpu.VMEM((per_w,), jnp.int32)]
            + [pltpu.VMEM((_CH, D), jnp.float32) for _ in range(_NB)]
            + [pltpu.SemaphoreType.DMA for _ in range(_NG * _NB + _NB)]
        ),
    )
    def run(table_hbm, idx_hbm, out_hbm, idx_v, *scratch):
        bufs = scratch[:_NB]
        gsem = scratch[_NB:_NB + _NG * _NB]
        wsem = scratch[_NB + _NG * _NB:]
        wid = lax.axis_index("subcore") * 2 + lax.axis_index("core")
        base = wid * per_w

        pltpu.sync_copy(idx_hbm.at[pl.ds(base, per_w)], idx_v)

        def gathers(g, b):
            return [
                pltpu.make_async_copy(
                    table_hbm.at[idx_v.at[pl.ds(g * _CH, _CH)]],
                    bufs[b],
                    gsem[b],
                )
            ]

        def write(g, b):
            return pltpu.make_async_copy(
                bufs[b], out_hbm.at[pl.ds(base + g * _CH, _CH)], wsem[b])

        for c in gathers(0, 0):
            c.start()

        @pl.loop(0, steps, step=_NB)
        def _(g0):
            for b in range(_NB):
                g = g0 + b
                bn = (b + 1) % _NB

                @pl.when(jnp.logical_and(g + 1 >= _NB, g + 1 < steps))
                def _():
                    write(g + 1 - _NB, bn).wait()

                @pl.when(g + 1 < steps)
                def _():
                    for c in gathers(g + 1, bn):
                        c.start()

                for c in gathers(g, b):
                    c.wait()
                write(g, b).start()

        for k in range(_NB):
            w = steps - _NB + k
            write(w, w % _NB).wait()

    return run(emb_weight, idx).reshape(B, L, D)


# speedup vs baseline: 1.0052x; 1.0011x over previous
import jax
import jax.numpy as jnp
from jax import lax
from jax.experimental import pallas as pl
from jax.experimental.pallas import tpu as pltpu
from jax.experimental.pallas import tpu_sc as plsc

_CH = 128
_NG = 1
_NB = 5


def kernel(x, emb_weight):
    B, L = x.shape
    N = B * L
    V, D = emb_weight.shape
    idx = x.reshape(N).astype(jnp.int32)

    NW = 32
    per_w = N // NW
    steps = per_w // _CH
    mesh = plsc.VectorSubcoreMesh(core_axis_name="core", subcore_axis_name="subcore")

    @pl.kernel(
        out_type=jax.ShapeDtypeStruct((N, D), emb_weight.dtype),
        mesh=mesh,
        scratch_types=(
            [pltpu.VMEM((per_w,), jnp.int32)]
            + [pltpu.VMEM((_CH, D), jnp.float32) for _ in range(_NB)]
            + [pltpu.SemaphoreType.DMA for _ in range(_NG * _NB + _NB)]
        ),
    )
    def run(table_hbm, idx_hbm, out_hbm, idx_v, *scratch):
        bufs = scratch[:_NB]
        gsem = scratch[_NB:_NB + _NG * _NB]
        wsem = scratch[_NB + _NG * _NB:]
        wid = lax.axis_index("subcore") * 2 + lax.axis_index("core")
        base = wid * per_w

        pltpu.sync_copy(idx_hbm.at[pl.ds(base, per_w)], idx_v)

        def gathers(g, b):
            return [
                pltpu.make_async_copy(
                    table_hbm.at[idx_v.at[pl.ds(g * _CH, _CH)]],
                    bufs[b],
                    gsem[b],
                )
            ]

        def write(g, b):
            return pltpu.make_async_copy(
                bufs[b], out_hbm.at[pl.ds(base + g * _CH, _CH)], wsem[b])

        for c in gathers(0, 0):
            c.start()
        for c in gathers(1, 1):
            c.start()

        @pl.loop(0, steps, step=_NB)
        def _(g0):
            for b in range(_NB):
                g = g0 + b
                bn = (b + 2) % _NB

                @pl.when(jnp.logical_and(g + 2 >= _NB, g + 2 < steps))
                def _():
                    write(g + 2 - _NB, bn).wait()

                @pl.when(g + 2 < steps)
                def _():
                    for c in gathers(g + 2, bn):
                        c.start()

                for c in gathers(g, b):
                    c.wait()
                write(g, b).start()

        for k in range(_NB):
            w = steps - _NB + k
            write(w, w % _NB).wait()

    return run(emb_weight, idx).reshape(B, L, D)
